# Initial kernel scaffold; baseline (speedup 1.0000x reference)
import jax
import jax.numpy as jnp
from jax.experimental import pallas as pl


def kernel(x, edge_index, edge_attr, W1, b1, W2, b2):
    raise NotImplementedError("write your pallas kernel here")


# speedup vs baseline: 10.7275x; 10.7275x over previous
import functools

import jax
import jax.numpy as jnp
from jax import lax
from jax.experimental import pallas as pl
from jax.experimental.pallas import tpu as pltpu
from jax.experimental.pallas import tpu_sc as plsc

NC = 2
NS = 16
NW = NC * NS
LANES = 16
EG = 128
SB = 16


def _deg_body(rows_pt, nzero,
              col2_hbm, ew2_hbm, out_hbm,
              acc, idxc, eww, zbuf):
    c = lax.axis_index("c")
    s = lax.axis_index("s")
    wid = s * NC + c

    for j in range(nzero // LANES):
        zbuf[pl.ds(j * LANES, LANES)] = jnp.zeros((LANES,), jnp.float32)
    pltpu.sync_copy(zbuf, acc.at[pl.ds(s * nzero, nzero)])

    rb = wid * rows_pt
    pltpu.sync_copy(col2_hbm.at[pl.ds(rb, rows_pt)], idxc)
    pltpu.sync_copy(ew2_hbm.at[pl.ds(rb, rows_pt)], eww)
    plsc.subcore_barrier()

    def ebody(j, carry):
        pltpu.sync_copy(eww.at[j], acc.at[idxc.at[j]], add=True)
        return carry

    lax.fori_loop(0, rows_pt, ebody, 0)
    plsc.subcore_barrier()

    pltpu.sync_copy(acc.at[pl.ds(s * nzero, nzero)], zbuf)
    pltpu.sync_copy(zbuf, out_hbm.at[c, pl.ds(s * nzero, nzero)])


def _sc_degree(col2, ew2, npad):
    nrows = col2.shape[0]
    rows_pt = nrows // NW
    nzero = npad // NS
    mesh = plsc.VectorSubcoreMesh(core_axis_name="c", subcore_axis_name="s")
    body = functools.partial(_deg_body, rows_pt, nzero)
    degp = pl.kernel(
        body,
        out_type=jax.ShapeDtypeStruct((NC, npad), jnp.float32),
        mesh=mesh,
        compiler_params=pltpu.CompilerParams(use_tc_tiling_on_sc=False),
        scratch_types=[
            pltpu.VMEM_SHARED((npad,), jnp.float32),
            pltpu.VMEM((rows_pt, EG), jnp.int32),
            pltpu.VMEM((rows_pt, EG), jnp.float32),
            pltpu.VMEM((nzero,), jnp.float32),
        ],
    )(col2, ew2)
    return degp[0], degp[1]


def _agg_body(h, rows_pt, rpt, qstep,
              row2_hbm, col2_hbm, ew2_hbm, hp_hbm, out_hbm,
              acc, idxr, idxc, eww, ring0):
    c = lax.axis_index("c")
    s = lax.axis_index("s")
    wid = s * NC + c

    for q in range(rpt // qstep):
        pltpu.sync_copy(hp_hbm.at[pl.ds(s * rpt + q * qstep, qstep)], ring0)
        pltpu.sync_copy(ring0, acc.at[pl.ds(s * rpt + q * qstep, qstep)])

    rb = wid * rows_pt
    plsc.subcore_barrier()

    def scale(ring, j):
        for g in range(EG // LANES):
            ev = eww[j, pl.ds(g * LANES, LANES)]
            for l in range(LANES):
                ws = ev[l]
                i = g * LANES + l
                for q in range(h // LANES):
                    ring[i, pl.ds(q * LANES, LANES)] = (
                        ring[i, pl.ds(q * LANES, LANES)] * ws)

    nsb = rows_pt // SB

    def sb_body(b, carry):
        rb2 = rb + b * SB
        pltpu.sync_copy(row2_hbm.at[pl.ds(rb2, SB)], idxr)
        pltpu.sync_copy(col2_hbm.at[pl.ds(rb2, SB)], idxc)
        pltpu.sync_copy(ew2_hbm.at[pl.ds(rb2, SB)], eww)

        def row_body(j, carry2):
            pltpu.sync_copy(hp_hbm.at[idxr.at[j]], ring0)
            scale(ring0, j)
            pltpu.sync_copy(ring0, acc.at[idxc.at[j]], add=True)
            return carry2

        lax.fori_loop(0, SB, row_body, 0)
        return carry

    lax.fori_loop(0, nsb, sb_body, 0)
    plsc.subcore_barrier()

    for q in range(rpt // qstep):
        pltpu.sync_copy(acc.at[pl.ds(s * rpt + q * qstep, qstep)], ring0)
        pltpu.sync_copy(ring0, out_hbm.at[c, pl.ds(s * rpt + q * qstep, qstep)])


def _sc_aggregate(row2, col2, ew2, hp):
    npad, h = hp.shape
    nrows = row2.shape[0]
    rows_pt = nrows // NW
    rpt = npad // NS
    qstep = EG
    mesh = plsc.VectorSubcoreMesh(core_axis_name="c", subcore_axis_name="s")
    body = functools.partial(_agg_body, h, rows_pt, rpt, qstep)
    agg = pl.kernel(
        body,
        out_type=jax.ShapeDtypeStruct((NC, npad, h), jnp.float32),
        mesh=mesh,
        compiler_params=pltpu.CompilerParams(use_tc_tiling_on_sc=False),
        scratch_types=[
            pltpu.VMEM_SHARED((npad, h), jnp.float32),
            pltpu.VMEM((SB, EG), jnp.int32),
            pltpu.VMEM((SB, EG), jnp.int32),
            pltpu.VMEM((SB, EG), jnp.float32),
            pltpu.VMEM((EG, h), jnp.float32),
        ],
    )(row2, col2, ew2, hp)
    return agg


def _prep_body(n, x_ref, w_ref, d0_ref, d1_ref, hp_ref, dinv_ref):
    deg = d0_ref[:n] + d1_ref[:n] + 1.0
    dinv = jnp.where(deg > 0, lax.rsqrt(deg), 0.0)
    dinv_ref[...] = dinv
    h = jnp.dot(x_ref[...], w_ref[...], preferred_element_type=jnp.float32)
    hp_ref[:n] = h * dinv
    hp_ref[n:] = jnp.zeros((hp_ref.shape[0] - n, hp_ref.shape[1]), jnp.float32)


def _tc_prep(x, w1, d0, d1, npad):
    n, f = x.shape
    h1 = w1.shape[1]
    return pl.pallas_call(
        functools.partial(_prep_body, n),
        out_shape=(jax.ShapeDtypeStruct((npad, h1), jnp.float32),
                   jax.ShapeDtypeStruct((n, 1), jnp.float32)),
    )(x, w1, d0, d1)


def _mid_body(n, agg_ref, hp_ref, dinv_ref, b_ref, w_ref, out_ref):
    comb = agg_ref[0, :n] + agg_ref[1, :n] - hp_ref[:n]
    u = jnp.maximum(comb * dinv_ref[...] + b_ref[...], 0.0)
    out_ref[:n] = jnp.dot(
        u, w_ref[...], preferred_element_type=jnp.float32) * dinv_ref[...]
    out_ref[n:] = jnp.zeros((out_ref.shape[0] - n, out_ref.shape[1]),
                            jnp.float32)


def _tc_mid(agg, hp, dinv, b, w2):
    npad, h1 = hp.shape
    n = dinv.shape[0]
    h2 = w2.shape[1]
    return pl.pallas_call(
        functools.partial(_mid_body, n),
        out_shape=jax.ShapeDtypeStruct((npad, h2), jnp.float32),
    )(agg, hp, dinv, b.reshape(1, h1), w2)


def _final_body(n, agg_ref, hp_ref, dinv_ref, b_ref, out_ref):
    comb = agg_ref[0, :n] + agg_ref[1, :n] - hp_ref[:n]
    out_ref[...] = jnp.maximum(comb * dinv_ref[...] + b_ref[...], 0.0)


def _tc_final(agg, hp, dinv, b):
    npad, h2 = hp.shape
    n = dinv.shape[0]
    return pl.pallas_call(
        functools.partial(_final_body, n),
        out_shape=jax.ShapeDtypeStruct((n, h2), jnp.float32),
    )(agg, hp, dinv, b.reshape(1, h2))


def kernel(x, edge_index, edge_attr, W1, b1, W2, b2):
    n = x.shape[0]
    e = edge_attr.shape[0]
    npad = ((n + 128 * NS - 1) // (128 * NS)) * (128 * NS)
    eblk = NW * EG * SB
    epad = ((e + eblk - 1) // eblk) * eblk
    pad = epad - e
    row2 = jnp.concatenate(
        [edge_index[0], jnp.zeros((pad,), edge_index.dtype)]).reshape(-1, EG)
    col2 = jnp.concatenate(
        [edge_index[1], jnp.zeros((pad,), edge_index.dtype)]).reshape(-1, EG)
    ew2 = jnp.concatenate(
        [edge_attr, jnp.zeros((pad,), edge_attr.dtype)]).reshape(-1, EG)

    deg0, deg1 = _sc_degree(col2, ew2, npad)
    d0 = deg0.reshape(npad, 1)
    d1 = deg1.reshape(npad, 1)

    h1p, dinv = _tc_prep(x, W1, d0, d1, npad)
    agg1 = _sc_aggregate(row2, col2, ew2, h1p)
    h2p = _tc_mid(agg1, h1p, dinv, b1, W2)
    agg2 = _sc_aggregate(row2, col2, ew2, h2p)
    out = _tc_final(agg2, h2p, dinv, b2)
    return out


# speedup vs baseline: 10.8660x; 1.0129x over previous
import functools

import jax
import jax.numpy as jnp
from jax import lax
from jax.experimental import pallas as pl
from jax.experimental.pallas import tpu as pltpu
from jax.experimental.pallas import tpu_sc as plsc

NC = 2
NS = 16
NW = NC * NS
LANES = 16
EG = 128
SB = 16


def _deg_body(rows_pt, nzero,
              col2_hbm, ew2_hbm, out_hbm,
              acc, idxc, eww, zbuf):
    c = lax.axis_index("c")
    s = lax.axis_index("s")
    wid = s * NC + c

    for j in range(nzero // LANES):
        zbuf[pl.ds(j * LANES, LANES)] = jnp.zeros((LANES,), jnp.float32)
    pltpu.sync_copy(zbuf, acc.at[pl.ds(s * nzero, nzero)])

    rb = wid * rows_pt
    pltpu.sync_copy(col2_hbm.at[pl.ds(rb, rows_pt)], idxc)
    pltpu.sync_copy(ew2_hbm.at[pl.ds(rb, rows_pt)], eww)
    plsc.subcore_barrier()

    def ebody(j, carry):
        pltpu.sync_copy(eww.at[j], acc.at[idxc.at[j]], add=True)
        return carry

    lax.fori_loop(0, rows_pt, ebody, 0)
    plsc.subcore_barrier()

    pltpu.sync_copy(acc.at[pl.ds(s * nzero, nzero)], zbuf)
    pltpu.sync_copy(zbuf, out_hbm.at[c, pl.ds(s * nzero, nzero)])


def _sc_degree(col2, ew2, npad):
    nrows = col2.shape[0]
    rows_pt = nrows // NW
    nzero = npad // NS
    mesh = plsc.VectorSubcoreMesh(core_axis_name="c", subcore_axis_name="s")
    body = functools.partial(_deg_body, rows_pt, nzero)
    degp = pl.kernel(
        body,
        out_type=jax.ShapeDtypeStruct((NC, npad), jnp.float32),
        mesh=mesh,
        compiler_params=pltpu.CompilerParams(use_tc_tiling_on_sc=False),
        scratch_types=[
            pltpu.VMEM_SHARED((npad,), jnp.float32),
            pltpu.VMEM((rows_pt, EG), jnp.int32),
            pltpu.VMEM((rows_pt, EG), jnp.float32),
            pltpu.VMEM((nzero,), jnp.float32),
        ],
    )(col2, ew2)
    return degp[0], degp[1]


def _agg_body(h, rows_pt, rpt, qstep,
              row2_hbm, col2_hbm, ew2_hbm, hp_hbm, out_hbm,
              acc, idxr, idxc, eww, ring0, ring1, sem0, sem1):
    c = lax.axis_index("c")
    s = lax.axis_index("s")
    wid = s * NC + c

    for q in range(rpt // qstep):
        pltpu.sync_copy(hp_hbm.at[pl.ds(s * rpt + q * qstep, qstep)], ring0)
        pltpu.sync_copy(ring0, acc.at[pl.ds(s * rpt + q * qstep, qstep)])

    rb = wid * rows_pt
    plsc.subcore_barrier()

    def scale(ring, j):
        for g in range(EG // LANES):
            ev = eww[j, pl.ds(g * LANES, LANES)]
            for l in range(LANES):
                ws = ev[l]
                i = g * LANES + l
                for q in range(h // LANES):
                    ring[i, pl.ds(q * LANES, LANES)] = (
                        ring[i, pl.ds(q * LANES, LANES)] * ws)

    nsb = rows_pt // SB

    def sb_body(b, carry):
        rb2 = rb + b * SB
        pltpu.sync_copy(row2_hbm.at[pl.ds(rb2, SB)], idxr)
        pltpu.sync_copy(col2_hbm.at[pl.ds(rb2, SB)], idxc)
        pltpu.sync_copy(ew2_hbm.at[pl.ds(rb2, SB)], eww)

        def pair_body(p, carry2):
            j0 = 2 * p
            j1 = j0 + 1
            cp0 = pltpu.async_copy(hp_hbm.at[idxr.at[j0]], ring0, sem0)
            cp1 = pltpu.async_copy(hp_hbm.at[idxr.at[j1]], ring1, sem1)
            cp0.wait()
            scale(ring0, j0)
            pltpu.sync_copy(ring0, acc.at[idxc.at[j0]], add=True)
            cp1.wait()
            scale(ring1, j1)
            pltpu.sync_copy(ring1, acc.at[idxc.at[j1]], add=True)
            return carry2

        lax.fori_loop(0, SB // 2, pair_body, 0)
        return carry

    lax.fori_loop(0, nsb, sb_body, 0)
    plsc.subcore_barrier()

    for q in range(rpt // qstep):
        pltpu.sync_copy(acc.at[pl.ds(s * rpt + q * qstep, qstep)], ring0)
        pltpu.sync_copy(ring0, out_hbm.at[c, pl.ds(s * rpt + q * qstep, qstep)])


def _sc_aggregate(row2, col2, ew2, hp):
    npad, h = hp.shape
    nrows = row2.shape[0]
    rows_pt = nrows // NW
    rpt = npad // NS
    qstep = EG
    mesh = plsc.VectorSubcoreMesh(core_axis_name="c", subcore_axis_name="s")
    body = functools.partial(_agg_body, h, rows_pt, rpt, qstep)
    agg = pl.kernel(
        body,
        out_type=jax.ShapeDtypeStruct((NC, npad, h), jnp.float32),
        mesh=mesh,
        compiler_params=pltpu.CompilerParams(use_tc_tiling_on_sc=False),
        scratch_types=[
            pltpu.VMEM_SHARED((npad, h), jnp.float32),
            pltpu.VMEM((SB, EG), jnp.int32),
            pltpu.VMEM((SB, EG), jnp.int32),
            pltpu.VMEM((SB, EG), jnp.float32),
            pltpu.VMEM((EG, h), jnp.float32),
            pltpu.VMEM((EG, h), jnp.float32),
            pltpu.SemaphoreType.DMA,
            pltpu.SemaphoreType.DMA,
        ],
    )(row2, col2, ew2, hp)
    return agg


def _prep_body(n, x_ref, w_ref, d0_ref, d1_ref, hp_ref, dinv_ref):
    deg = d0_ref[:n] + d1_ref[:n] + 1.0
    dinv = jnp.where(deg > 0, lax.rsqrt(deg), 0.0)
    dinv_ref[...] = dinv
    h = jnp.dot(x_ref[...], w_ref[...], preferred_element_type=jnp.float32)
    hp_ref[:n] = h * dinv
    hp_ref[n:] = jnp.zeros((hp_ref.shape[0] - n, hp_ref.shape[1]), jnp.float32)


def _tc_prep(x, w1, d0, d1, npad):
    n, f = x.shape
    h1 = w1.shape[1]
    return pl.pallas_call(
        functools.partial(_prep_body, n),
        out_shape=(jax.ShapeDtypeStruct((npad, h1), jnp.float32),
                   jax.ShapeDtypeStruct((n, 1), jnp.float32)),
    )(x, w1, d0, d1)


def _mid_body(n, agg_ref, hp_ref, dinv_ref, b_ref, w_ref, out_ref):
    comb = agg_ref[0, :n] + agg_ref[1, :n] - hp_ref[:n]
    u = jnp.maximum(comb * dinv_ref[...] + b_ref[...], 0.0)
    out_ref[:n] = jnp.dot(
        u, w_ref[...], preferred_element_type=jnp.float32) * dinv_ref[...]
    out_ref[n:] = jnp.zeros((out_ref.shape[0] - n, out_ref.shape[1]),
                            jnp.float32)


def _tc_mid(agg, hp, dinv, b, w2):
    npad, h1 = hp.shape
    n = dinv.shape[0]
    h2 = w2.shape[1]
    return pl.pallas_call(
        functools.partial(_mid_body, n),
        out_shape=jax.ShapeDtypeStruct((npad, h2), jnp.float32),
    )(agg, hp, dinv, b.reshape(1, h1), w2)


def _final_body(n, agg_ref, hp_ref, dinv_ref, b_ref, out_ref):
    comb = agg_ref[0, :n] + agg_ref[1, :n] - hp_ref[:n]
    out_ref[...] = jnp.maximum(comb * dinv_ref[...] + b_ref[...], 0.0)


def _tc_final(agg, hp, dinv, b):
    npad, h2 = hp.shape
    n = dinv.shape[0]
    return pl.pallas_call(
        functools.partial(_final_body, n),
        out_shape=jax.ShapeDtypeStruct((n, h2), jnp.float32),
    )(agg, hp, dinv, b.reshape(1, h2))


def kernel(x, edge_index, edge_attr, W1, b1, W2, b2):
    n = x.shape[0]
    e = edge_attr.shape[0]
    npad = ((n + 128 * NS - 1) // (128 * NS)) * (128 * NS)
    eblk = NW * EG * SB
    epad = ((e + eblk - 1) // eblk) * eblk
    pad = epad - e
    row2 = jnp.concatenate(
        [edge_index[0], jnp.zeros((pad,), edge_index.dtype)]).reshape(-1, EG)
    col2 = jnp.concatenate(
        [edge_index[1], jnp.zeros((pad,), edge_index.dtype)]).reshape(-1, EG)
    ew2 = jnp.concatenate(
        [edge_attr, jnp.zeros((pad,), edge_attr.dtype)]).reshape(-1, EG)

    deg0, deg1 = _sc_degree(col2, ew2, npad)
    d0 = deg0.reshape(npad, 1)
    d1 = deg1.reshape(npad, 1)

    h1p, dinv = _tc_prep(x, W1, d0, d1, npad)
    agg1 = _sc_aggregate(row2, col2, ew2, h1p)
    h2p = _tc_mid(agg1, h1p, dinv, b1, W2)
    agg2 = _sc_aggregate(row2, col2, ew2, h2p)
    out = _tc_final(agg2, h2p, dinv, b2)
    return out
